# baseline (device time: 16196 ns/iter reference)
import jax
import jax.numpy as jnp
from jax import lax
from jax.experimental import pallas as pl
from jax.experimental.pallas import tpu as pltpu

N_DEV = 4
B, SQ, SKV = 2, 128, 128
D_MODEL = 512
HQ_LOCAL, DH = 4, 64


def kernel(x, Wq, K_ext, V_ext, Wo):
    KW = HQ_LOCAL * DH
    K_loc = K_ext.reshape(B, SKV, 16 * DH).astype(jnp.bfloat16)
    V_loc = V_ext.reshape(B, SKV, 16 * DH).astype(jnp.bfloat16)
    x = x.astype(jnp.bfloat16)
    Wq = Wq.astype(jnp.bfloat16)
    Wo = Wo.astype(jnp.bfloat16)

    def body(x_ref, wq_ref, k_ref, v_ref, wo_ref, out_ref,
             s1x_ref, r1x_ref, s1y_ref, r1y_ref,
             s2x_ref, r2x_ref, s2y_ref, r2y_ref, send_sems, recv_sems):
        my_pos = lax.axis_index("i")
        pa = my_pos ^ 1
        pb = 3 - my_pos

        barrier_sem = pltpu.get_barrier_semaphore()
        for nbr in (pa, pb):
            pl.semaphore_signal(
                barrier_sem, inc=1,
                device_id=(nbr,), device_id_type=pl.DeviceIdType.MESH,
            )
        pl.semaphore_wait(barrier_sem, 2)

        x2 = x_ref[...].reshape(B * SQ, D_MODEL)
        q2 = jnp.dot(
            x2, wq_ref[...], preferred_element_type=jnp.float32
        ).astype(jnp.bfloat16)

        ri = lax.broadcasted_iota(jnp.int32, (SQ, SKV), 0)
        ci = lax.broadcasted_iota(jnp.int32, (SQ, SKV), 1)
        mask = ((ri // 64) == (ci // 64)).astype(jnp.float32)

        ctx_rows = []
        for b in range(B):
            kb = k_ref[b, :, pl.ds(my_pos * KW, KW)]
            vb = v_ref[b, :, pl.ds(my_pos * KW, KW)]
            heads = []
            for h in range(HQ_LOCAL):
                qbh = q2[b * SQ:(b + 1) * SQ, h * DH:(h + 1) * DH]
                kbh = kb[:, h * DH:(h + 1) * DH]
                vbh = vb[:, h * DH:(h + 1) * DH]
                s = lax.dot_general(
                    qbh, kbh, (((1,), (1,)), ((), ())),
                    preferred_element_type=jnp.float32,
                ) * 0.125
                e = jnp.exp(s) * mask
                w = (e / jnp.sum(e, axis=-1, keepdims=True)).astype(jnp.bfloat16)
                heads.append(
                    jnp.dot(w, vbh, preferred_element_type=jnp.float32)
                    .astype(jnp.bfloat16)
                )
            ctx_rows.append(jnp.concatenate(heads, axis=1))
        ctx2 = jnp.concatenate(ctx_rows, axis=0)

        partial = jnp.dot(ctx2, wo_ref[...], preferred_element_type=jnp.float32)
        partial = partial.reshape(B, SQ, D_MODEL)

        HALF = D_MODEL // 2

        def xchg(s_ref, r_ref, sem_idx, partner):
            return pltpu.make_async_remote_copy(
                src_ref=s_ref, dst_ref=r_ref,
                send_sem=send_sems.at[sem_idx], recv_sem=recv_sems.at[sem_idx],
                device_id=(partner,), device_id_type=pl.DeviceIdType.MESH,
            )

        s1x_ref[...] = partial[:, :, :HALF].astype(jnp.bfloat16)
        s1y_ref[...] = partial[:, :, HALF:].astype(jnp.bfloat16)
        rx1 = xchg(s1x_ref, r1x_ref, 0, pa)
        ry1 = xchg(s1y_ref, r1y_ref, 1, pb)
        rx1.start()
        ry1.start()
        out_ref[...] = partial

        rx1.wait()
        accx = out_ref[:, :, :HALF] + r1x_ref[...].astype(jnp.float32)
        s2x_ref[...] = accx.astype(jnp.bfloat16)
        rx2 = xchg(s2x_ref, r2x_ref, 2, pb)
        rx2.start()

        ry1.wait()
        accy = out_ref[:, :, HALF:] + r1y_ref[...].astype(jnp.float32)
        s2y_ref[...] = accy.astype(jnp.bfloat16)
        ry2 = xchg(s2y_ref, r2y_ref, 3, pa)
        ry2.start()

        out_ref[:, :, :HALF] = accx
        out_ref[:, :, HALF:] = accy
        rx2.wait()
        out_ref[:, :, :HALF] += r2x_ref[...].astype(jnp.float32)
        ry2.wait()
        out_ref[:, :, HALF:] += r2y_ref[...].astype(jnp.float32)

    comm_shape = (B, SQ, D_MODEL // 2)
    return pl.pallas_call(
        body,
        out_shape=jax.ShapeDtypeStruct((B, SQ, D_MODEL), jnp.float32),
        in_specs=[pl.BlockSpec(memory_space=pltpu.VMEM)] * 5,
        out_specs=pl.BlockSpec(memory_space=pltpu.VMEM),
        scratch_shapes=[pltpu.VMEM(comm_shape, jnp.bfloat16)] * 8 + [
            pltpu.SemaphoreType.DMA((4,)),
            pltpu.SemaphoreType.DMA((4,)),
        ],
        compiler_params=pltpu.CompilerParams(collective_id=0),
    )(x, Wq, K_loc, V_loc, Wo)


# device time: 13779 ns/iter; 1.1754x vs baseline; 1.1754x over previous
import jax
import jax.numpy as jnp
from jax import lax
from jax.experimental import pallas as pl
from jax.experimental.pallas import tpu as pltpu

N_DEV = 4
B, SQ, SKV = 2, 128, 128
D_MODEL = 512
HQ_LOCAL, DH = 4, 64


def kernel(x, Wq, K_ext, V_ext, Wo):
    KW = HQ_LOCAL * DH
    K_loc = K_ext.reshape(B, SKV, 16 * DH)
    V_loc = V_ext.reshape(B, SKV, 16 * DH)

    HALF = D_MODEL // 2

    def body(x_ref, wq_ref, k_ref, v_ref, wo_ref, out_ref,
             comm_ref, send_sems, recv_sems):
        my_pos = lax.axis_index("i")
        pa = my_pos ^ 1
        pb = 3 - my_pos

        barrier_sem = pltpu.get_barrier_semaphore()
        for nbr in (pa, pb):
            pl.semaphore_signal(
                barrier_sem, inc=1,
                device_id=(nbr,), device_id_type=pl.DeviceIdType.MESH,
            )
        pl.semaphore_wait(barrier_sem, 2)

        def xchg(src_slot, dst_slot, sem_idx, partner):
            return pltpu.make_async_remote_copy(
                src_ref=comm_ref.at[src_slot], dst_ref=comm_ref.at[dst_slot],
                send_sem=send_sems.at[sem_idx], recv_sem=recv_sems.at[sem_idx],
                device_id=(partner,), device_id_type=pl.DeviceIdType.MESH,
            )

        r1_partner = (pa, pb)
        r2_partner = (pb, pa)

        x2 = x_ref[...].reshape(B * SQ, D_MODEL)
        q2 = jnp.dot(x2, wq_ref[...], preferred_element_type=jnp.float32)

        ri = lax.broadcasted_iota(jnp.int32, (SQ, SKV), 0)
        ci = lax.broadcasted_iota(jnp.int32, (SQ, SKV), 1)
        mask = ((ri // 64) == (ci // 64)).astype(jnp.float32)

        r1 = {}
        for b in range(B):
            kb = k_ref[b, :, pl.ds(my_pos * KW, KW)]
            vb = v_ref[b, :, pl.ds(my_pos * KW, KW)]
            heads = []
            for h in range(HQ_LOCAL):
                qbh = q2[b * SQ:(b + 1) * SQ, h * DH:(h + 1) * DH]
                kbh = kb[:, h * DH:(h + 1) * DH]
                vbh = vb[:, h * DH:(h + 1) * DH]
                s = lax.dot_general(
                    qbh, kbh, (((1,), (1,)), ((), ())),
                    preferred_element_type=jnp.float32,
                ) * 0.125
                e = jnp.exp(s) * mask
                w = e / jnp.sum(e, axis=-1, keepdims=True)
                heads.append(jnp.dot(w, vbh, preferred_element_type=jnp.float32))
            ctx_b = jnp.concatenate(heads, axis=1)
            partial_b = jnp.dot(
                ctx_b, wo_ref[...], preferred_element_type=jnp.float32
            )
            for half in range(2):
                comm_ref[b * 2 + half] = (
                    partial_b[:, half * HALF:(half + 1) * HALF]
                    .astype(jnp.bfloat16))
                r1[b, half] = xchg(
                    b * 2 + half, 4 + b * 2 + half, b * 2 + half,
                    r1_partner[half])
                r1[b, half].start()
            out_ref[b] = partial_b

        r2 = {}
        for b in range(B):
            for half in range(2):
                q_idx = b * 2 + half
                r1[b, half].wait()
                acc = (out_ref[b, :, half * HALF:(half + 1) * HALF]
                       + comm_ref[4 + q_idx].astype(jnp.float32))
                comm_ref[8 + q_idx] = acc.astype(jnp.bfloat16)
                r2[b, half] = xchg(8 + q_idx, 12 + q_idx, 4 + q_idx,
                                   r2_partner[half])
                r2[b, half].start()
                out_ref[b, :, half * HALF:(half + 1) * HALF] = acc

        for b in range(B):
            for half in range(2):
                q_idx = b * 2 + half
                r2[b, half].wait()
                out_ref[b, :, half * HALF:(half + 1) * HALF] += (
                    comm_ref[12 + q_idx].astype(jnp.float32))

    return pl.pallas_call(
        body,
        out_shape=jax.ShapeDtypeStruct((B, SQ, D_MODEL), jnp.float32),
        in_specs=[pl.BlockSpec(memory_space=pltpu.VMEM)] * 5,
        out_specs=pl.BlockSpec(memory_space=pltpu.VMEM),
        scratch_shapes=[
            pltpu.VMEM((16, SQ, HALF), jnp.bfloat16),
            pltpu.SemaphoreType.DMA((8,)),
            pltpu.SemaphoreType.DMA((8,)),
        ],
        compiler_params=pltpu.CompilerParams(collective_id=0),
    )(x, Wq, K_loc, V_loc, Wo)
